# Initial kernel scaffold; baseline (speedup 1.0000x reference)
import functools

import jax
import jax.numpy as jnp
from jax import lax
from jax.experimental import pallas as pl
from jax.experimental.pallas import tpu as pltpu
from jax.experimental.pallas import tpu_sc as plsc

N = 10000
D = 128
H = 256
OUT = 2
E = 320000

NC = 2
NS = 16
NW = NC * NS
EB = 128
TB = (E + NW * EB - 1) // (NW * EB)
EPT = TB * EB
E_PAD = EPT * NW
N_PAD = 10240
RPT = N_PAD // NS
ROW_OUT = N // NS
CW = 16

_MESH = plsc.VectorSubcoreMesh(core_axis_name="c", subcore_axis_name="s")


@functools.partial(
    pl.kernel,
    out_type=[
        jax.ShapeDtypeStruct((NC, N, D), jnp.float32),
        jax.ShapeDtypeStruct((NC, N, CW), jnp.float32),
    ],
    mesh=_MESH,
    scratch_types=[
        pltpu.VMEM((EB,), jnp.int32),
        pltpu.VMEM((EB,), jnp.int32),
        pltpu.VMEM((EB, D), jnp.float32),
        pltpu.VMEM((EB, CW), jnp.float32),
        pltpu.VMEM_SHARED((N_PAD, D), jnp.float32),
        pltpu.VMEM_SHARED((N_PAD, CW), jnp.float32),
        pltpu.SemaphoreType.DMA,
    ],
)
def _sc_segsum_wide(x_hbm, src_hbm, dst_hbm, zrow_hbm, zcnt_hbm, ones_hbm,
                    aggr_out, cnt_out, src_v, dst_v, rows_v, ones_v,
                    acc, cacc, sem):
    c = lax.axis_index("c")
    s = lax.axis_index("s")
    wid = c * NS + s
    pltpu.sync_copy(zrow_hbm, acc.at[pl.ds(s * RPT, RPT)])
    pltpu.sync_copy(zcnt_hbm, cacc.at[pl.ds(s * RPT, RPT)])
    pltpu.sync_copy(ones_hbm, ones_v)
    plsc.subcore_barrier()

    base = wid * EPT

    def body(b, carry):
        off = base + b * EB
        pltpu.sync_copy(src_hbm.at[pl.ds(off, EB)], src_v)
        pltpu.sync_copy(dst_hbm.at[pl.ds(off, EB)], dst_v)
        pltpu.async_copy(x_hbm.at[src_v], rows_v, sem).wait()
        pltpu.sync_copy(rows_v, acc.at[dst_v], add=True)
        pltpu.sync_copy(ones_v, cacc.at[dst_v], add=True)
        return carry

    lax.fori_loop(0, TB, body, 0)
    plsc.subcore_barrier()

    r0 = s * ROW_OUT
    pltpu.sync_copy(acc.at[pl.ds(r0, ROW_OUT)], aggr_out.at[c, pl.ds(r0, ROW_OUT)])
    pltpu.sync_copy(cacc.at[pl.ds(r0, ROW_OUT)], cnt_out.at[c, pl.ds(r0, ROW_OUT)])


@functools.partial(
    pl.kernel,
    out_type=[jax.ShapeDtypeStruct((NC, N, CW), jnp.float32)],
    mesh=_MESH,
    scratch_types=[
        pltpu.VMEM((EB,), jnp.int32),
        pltpu.VMEM((EB,), jnp.int32),
        pltpu.VMEM((EB, CW), jnp.float32),
        pltpu.VMEM_SHARED((N_PAD, CW), jnp.float32),
        pltpu.SemaphoreType.DMA,
    ],
)
def _sc_segsum_narrow(p_hbm, src_hbm, dst_hbm, zcnt_hbm,
                      acc_out, src_v, dst_v, rows_v, acc, sem):
    c = lax.axis_index("c")
    s = lax.axis_index("s")
    wid = c * NS + s
    pltpu.sync_copy(zcnt_hbm, acc.at[pl.ds(s * RPT, RPT)])
    plsc.subcore_barrier()

    base = wid * EPT

    def body(b, carry):
        off = base + b * EB
        pltpu.sync_copy(src_hbm.at[pl.ds(off, EB)], src_v)
        pltpu.sync_copy(dst_hbm.at[pl.ds(off, EB)], dst_v)
        pltpu.async_copy(p_hbm.at[src_v], rows_v, sem).wait()
        pltpu.sync_copy(rows_v, acc.at[dst_v], add=True)
        return carry

    lax.fori_loop(0, TB, body, 0)
    plsc.subcore_barrier()

    r0 = s * ROW_OUT
    pltpu.sync_copy(acc.at[pl.ds(r0, ROW_OUT)], acc_out.at[c, pl.ds(r0, ROW_OUT)])


RB = 500


def _tc_mid_body(aggr_ref, cnt_ref, x_ref, wl1_ref, wc_ref, bc_ref, g_ref,
                 bln_ref, wl2_ref, wr2_ref, bl2_ref, p_ref, r_ref):
    cnt = jnp.maximum(cnt_ref[0, :, 0:1] + cnt_ref[1, :, 0:1], 1.0)
    mean = (aggr_ref[0] + aggr_ref[1]) / cnt
    x1 = (jnp.dot(mean, wl1_ref[...], preferred_element_type=jnp.float32)
          + jnp.dot(x_ref[...], wc_ref[...], preferred_element_type=jnp.float32)
          + bc_ref[...])
    mu = jnp.mean(x1, axis=-1, keepdims=True)
    var = jnp.mean((x1 - mu) * (x1 - mu), axis=-1, keepdims=True)
    xn = (x1 - mu) * lax.rsqrt(var + 1e-5) * g_ref[...] + bln_ref[...]
    h = jnp.where(xn > 0, xn, jnp.expm1(xn))
    p_ref[...] = jnp.dot(h, wl2_ref[...], preferred_element_type=jnp.float32)
    r_ref[...] = (jnp.dot(h, wr2_ref[...], preferred_element_type=jnp.float32)
                  + bl2_ref[...])


def _tc_out_body(acc2_ref, cnt_ref, r_ref, o_ref):
    cnt = jnp.maximum(cnt_ref[0, :, 0:1] + cnt_ref[1, :, 0:1], 1.0)
    o_ref[...] = (acc2_ref[0] + acc2_ref[1]) / cnt + r_ref[...]


def kernel(x, edge_index, Wl1, bl1, Wr1, Ws, bs, g1, b1, Wl2, bl2, Wr2):
    src = edge_index[0]
    dst = edge_index[1]
    pad = E_PAD - E
    srcp = jnp.concatenate([src, jnp.zeros((pad,), jnp.int32)])
    dstp = jnp.concatenate([dst, jnp.full((pad,), N, jnp.int32)])
    zrow = jnp.zeros((RPT, D), jnp.float32)
    zcnt = jnp.zeros((RPT, CW), jnp.float32)
    ones = jnp.ones((EB, CW), jnp.float32)

    aggr_p, cnt_p = _sc_segsum_wide(x, srcp, dstp, zrow, zcnt, ones)

    wl1t = Wl1.T
    wct = (Wr1 + Ws).T
    bc = (bl1 + bs).reshape(1, H)
    g = g1.reshape(1, H)
    bln = b1.reshape(1, H)
    wl2t = jnp.zeros((H, CW), jnp.float32).at[:, :OUT].set(Wl2.T)
    wr2t = jnp.zeros((H, CW), jnp.float32).at[:, :OUT].set(Wr2.T)
    bl2p = jnp.zeros((1, CW), jnp.float32).at[0, :OUT].set(bl2)

    grid = (N // RB,)
    p, r = pl.pallas_call(
        _tc_mid_body,
        grid=grid,
        in_specs=[
            pl.BlockSpec((NC, RB, D), lambda i: (0, i, 0)),
            pl.BlockSpec((NC, RB, CW), lambda i: (0, i, 0)),
            pl.BlockSpec((RB, D), lambda i: (i, 0)),
            pl.BlockSpec((D, H), lambda i: (0, 0)),
            pl.BlockSpec((D, H), lambda i: (0, 0)),
            pl.BlockSpec((1, H), lambda i: (0, 0)),
            pl.BlockSpec((1, H), lambda i: (0, 0)),
            pl.BlockSpec((1, H), lambda i: (0, 0)),
            pl.BlockSpec((H, CW), lambda i: (0, 0)),
            pl.BlockSpec((H, CW), lambda i: (0, 0)),
            pl.BlockSpec((1, CW), lambda i: (0, 0)),
        ],
        out_specs=[
            pl.BlockSpec((RB, CW), lambda i: (i, 0)),
            pl.BlockSpec((RB, CW), lambda i: (i, 0)),
        ],
        out_shape=[
            jax.ShapeDtypeStruct((N, CW), jnp.float32),
            jax.ShapeDtypeStruct((N, CW), jnp.float32),
        ],
    )(aggr_p, cnt_p, x, wl1t, wct, bc, g, bln, wl2t, wr2t, bl2p)

    (acc2_p,) = _sc_segsum_narrow(p, srcp, dstp, zcnt)

    out16 = pl.pallas_call(
        _tc_out_body,
        out_shape=jax.ShapeDtypeStruct((N, CW), jnp.float32),
    )(acc2_p, cnt_p, r)

    return out16[:, :OUT]


# speedup vs baseline: 5.0963x; 5.0963x over previous
import functools

import jax
import jax.numpy as jnp
from jax import lax
from jax.experimental import pallas as pl
from jax.experimental.pallas import tpu as pltpu
from jax.experimental.pallas import tpu_sc as plsc

N = 10000
D = 128
H = 256
OUT = 2
E = 320000

NC = 2
NS = 16
NW = NC * NS
L = 16
EB = 128
TB = (E + NW * EB - 1) // (NW * EB)
EPT = TB * EB
E_PAD = EPT * NW
N_PAD = 10240
RPT = N_PAD // NS
CW = 16

_MESH = plsc.VectorSubcoreMesh(core_axis_name="c", subcore_axis_name="s",
                               num_cores=NC, num_subcores=NS)


def _make_segsum(with_counts):
    out_type = [jax.ShapeDtypeStruct((NC, N_PAD, D), jnp.float32)]
    scratch = [
        pltpu.VMEM((EB,), jnp.int32),
        pltpu.VMEM((EB,), jnp.int32),
        pltpu.VMEM((EB, D), jnp.float32),
        pltpu.VMEM_SHARED((N_PAD, D), jnp.float32),
        pltpu.SemaphoreType.DMA,
    ]
    if with_counts:
        out_type.append(jax.ShapeDtypeStruct((NW, N_PAD), jnp.float32))
        scratch.append(pltpu.VMEM((N_PAD,), jnp.float32))

    def body(*refs):
        if with_counts:
            (x_hbm, src_hbm, dst_hbm, zrow_hbm, zcnt_hbm,
             aggr_out, cnt_out, src_v, dst_v, rows_v, acc, sem, cnt_v) = refs
        else:
            (x_hbm, src_hbm, dst_hbm, zrow_hbm,
             aggr_out, src_v, dst_v, rows_v, acc, sem) = refs
        c = lax.axis_index("c")
        s = lax.axis_index("s")
        wid = c * NS + s
        pltpu.sync_copy(zrow_hbm, acc.at[pl.ds(s * RPT, RPT)])
        if with_counts:
            pltpu.sync_copy(zcnt_hbm, cnt_v)
        plsc.subcore_barrier()

        base = wid * EPT
        ones16 = jnp.ones((L,), jnp.float32)

        def step(b, carry):
            off = base + b * EB
            pltpu.sync_copy(src_hbm.at[pl.ds(off, EB)], src_v)
            pltpu.sync_copy(dst_hbm.at[pl.ds(off, EB)], dst_v)
            pltpu.async_copy(x_hbm.at[src_v], rows_v, sem).wait()
            pltpu.sync_copy(rows_v, acc.at[dst_v], add=True)
            if with_counts:
                for j in range(EB // L):
                    dvec = dst_v[pl.ds(j * L, L)]
                    plsc.addupdate_scatter(cnt_v, [dvec], ones16)
            return carry

        lax.fori_loop(0, TB, step, 0)
        plsc.subcore_barrier()

        r0 = s * RPT
        pltpu.sync_copy(acc.at[pl.ds(r0, RPT)], aggr_out.at[c, pl.ds(r0, RPT)])
        if with_counts:
            pltpu.sync_copy(cnt_v, cnt_out.at[wid])

    return functools.partial(
        pl.kernel, out_type=out_type, mesh=_MESH, scratch_types=scratch,
        compiler_params=pltpu.CompilerParams(needs_layout_passes=False))(body)


_segsum_counts = _make_segsum(True)
_segsum_plain = _make_segsum(False)

RB = 2048


def _tc_mid_body(aggr_ref, cnt_ref, x_ref, wl1_ref, wc_ref, bc_ref, g_ref,
                 bln_ref, wl2_ref, wr2_ref, bl2_ref, p_ref, r_ref):
    cnt = jnp.maximum(jnp.sum(cnt_ref[...], axis=0), 1.0).reshape(-1, 1)
    mean = (aggr_ref[0] + aggr_ref[1]) / cnt
    x1 = (jnp.dot(mean, wl1_ref[...], preferred_element_type=jnp.float32)
          + jnp.dot(x_ref[...], wc_ref[...], preferred_element_type=jnp.float32)
          + bc_ref[...])
    mu = jnp.mean(x1, axis=-1, keepdims=True)
    var = jnp.mean((x1 - mu) * (x1 - mu), axis=-1, keepdims=True)
    xn = (x1 - mu) * lax.rsqrt(var + 1e-5) * g_ref[...] + bln_ref[...]
    h = jnp.where(xn > 0, xn, jnp.exp(jnp.minimum(xn, 0.0)) - 1.0)
    p_ref[...] = jnp.dot(h, wl2_ref[...], preferred_element_type=jnp.float32)
    r_ref[...] = (jnp.dot(h, wr2_ref[...], preferred_element_type=jnp.float32)
                  + bl2_ref[...])


def _tc_out_body(acc2_ref, cnt_ref, r_ref, o_ref):
    cnt = jnp.maximum(jnp.sum(cnt_ref[...], axis=0), 1.0).reshape(-1, 1)
    o_ref[...] = (acc2_ref[0, :, :CW] + acc2_ref[1, :, :CW]) / cnt + r_ref[...]


def kernel(x, edge_index, Wl1, bl1, Wr1, Ws, bs, g1, b1, Wl2, bl2, Wr2):
    src = edge_index[0]
    dst = edge_index[1]
    pad = E_PAD - E
    srcp = jnp.concatenate([src, jnp.zeros((pad,), jnp.int32)])
    dstp = jnp.concatenate([dst, jnp.full((pad,), N, jnp.int32)])
    zrow = jnp.zeros((RPT, D), jnp.float32)
    zcnt = jnp.zeros((N_PAD,), jnp.float32)

    aggr_p, cnt_p = _segsum_counts(x, srcp, dstp, zrow, zcnt)

    xp = jnp.concatenate([x, jnp.zeros((N_PAD - N, D), jnp.float32)])
    wl1t = Wl1.T
    wct = (Wr1 + Ws).T
    bc = (bl1 + bs).reshape(1, H)
    g = g1.reshape(1, H)
    bln = b1.reshape(1, H)
    wl2t = jnp.zeros((H, D), jnp.float32).at[:, :OUT].set(Wl2.T)
    wr2t = jnp.zeros((H, CW), jnp.float32).at[:, :OUT].set(Wr2.T)
    bl2p = jnp.zeros((1, CW), jnp.float32).at[0, :OUT].set(bl2)

    grid = (N_PAD // RB,)
    p, r = pl.pallas_call(
        _tc_mid_body,
        grid=grid,
        in_specs=[
            pl.BlockSpec((NC, RB, D), lambda i: (0, i, 0)),
            pl.BlockSpec((NW, RB), lambda i: (0, i)),
            pl.BlockSpec((RB, D), lambda i: (i, 0)),
            pl.BlockSpec((D, H), lambda i: (0, 0)),
            pl.BlockSpec((D, H), lambda i: (0, 0)),
            pl.BlockSpec((1, H), lambda i: (0, 0)),
            pl.BlockSpec((1, H), lambda i: (0, 0)),
            pl.BlockSpec((1, H), lambda i: (0, 0)),
            pl.BlockSpec((H, D), lambda i: (0, 0)),
            pl.BlockSpec((H, CW), lambda i: (0, 0)),
            pl.BlockSpec((1, CW), lambda i: (0, 0)),
        ],
        out_specs=[
            pl.BlockSpec((RB, D), lambda i: (i, 0)),
            pl.BlockSpec((RB, CW), lambda i: (i, 0)),
        ],
        out_shape=[
            jax.ShapeDtypeStruct((N_PAD, D), jnp.float32),
            jax.ShapeDtypeStruct((N_PAD, CW), jnp.float32),
        ],
    )(aggr_p, cnt_p, xp, wl1t, wct, bc, g, bln, wl2t, wr2t, bl2p)

    (acc2_p,) = _segsum_plain(p, srcp, dstp, zrow)

    out16 = pl.pallas_call(
        _tc_out_body,
        grid=grid,
        in_specs=[
            pl.BlockSpec((NC, RB, D), lambda i: (0, i, 0)),
            pl.BlockSpec((NW, RB), lambda i: (0, i)),
            pl.BlockSpec((RB, CW), lambda i: (i, 0)),
        ],
        out_specs=pl.BlockSpec((RB, CW), lambda i: (i, 0)),
        out_shape=jax.ShapeDtypeStruct((N_PAD, CW), jnp.float32),
    )(acc2_p, cnt_p, r)

    return out16[:N, :OUT]


# speedup vs baseline: 7.2732x; 1.4272x over previous
import functools

import jax
import jax.numpy as jnp
from jax import lax
from jax.experimental import pallas as pl
from jax.experimental.pallas import tpu as pltpu
from jax.experimental.pallas import tpu_sc as plsc

N = 10000
D = 128
H = 256
OUT = 2
E = 320000

NC = 2
NS = 16
NW = NC * NS
L = 16
EB = 128
NBUF = 4
TB = 80
G = TB // NBUF
EPT = TB * EB
E_PAD = EPT * NW
N_PAD = 10240
RPT = N_PAD // NS
CW = 16

_MESH = plsc.VectorSubcoreMesh(core_axis_name="c", subcore_axis_name="s",
                               num_cores=NC, num_subcores=NS)
_SC_PARAMS = pltpu.CompilerParams(needs_layout_passes=False)


QN = 4
G4 = TB // QN


@functools.partial(
    pl.kernel,
    out_type=[
        jax.ShapeDtypeStruct((NC, N_PAD, D), jnp.float32),
        jax.ShapeDtypeStruct((NW, N_PAD), jnp.float32),
    ],
    mesh=_MESH,
    scratch_types=[
        pltpu.VMEM((QN, EB), jnp.int32),
        pltpu.VMEM((QN, EB), jnp.int32),
        pltpu.VMEM((2, EB, D), jnp.float32),
        pltpu.VMEM((N_PAD,), jnp.float32),
        pltpu.VMEM_SHARED((N_PAD, D), jnp.float32),
        [pltpu.SemaphoreType.DMA] * QN,
        [pltpu.SemaphoreType.DMA] * 2,
        [pltpu.SemaphoreType.DMA] * 2,
    ],
    compiler_params=_SC_PARAMS,
)
def _sc_segsum(x_hbm, src_hbm, dst_hbm, zrow_hbm, zcnt_hbm,
               aggr_out, cnt_out, sidx, didx, rows, cnt_v, acc,
               sem_i, sem_g, sem_s):
    c = lax.axis_index("c")
    s = lax.axis_index("s")
    wid = c * NS + s
    pltpu.sync_copy(zrow_hbm, acc.at[pl.ds(s * RPT, RPT)])
    pltpu.sync_copy(zcnt_hbm, cnt_v)
    plsc.subcore_barrier()

    ones16 = jnp.ones((L,), jnp.float32)

    def idx_load(q, b):
        pltpu.async_copy(src_hbm.at[wid, b], sidx.at[q], sem_i[q])
        pltpu.async_copy(dst_hbm.at[wid, b], didx.at[q], sem_i[q])

    def idx_wait(q):
        pltpu.make_async_copy(src_hbm.at[wid, 0], sidx.at[q], sem_i[q]).wait()
        pltpu.make_async_copy(dst_hbm.at[wid, 0], didx.at[q], sem_i[q]).wait()

    def gather(k, q):
        pltpu.async_copy(x_hbm.at[sidx.at[q]], rows.at[k], sem_g[k])

    def gather_wait(k):
        pltpu.make_async_copy(x_hbm.at[sidx.at[0]], rows.at[k],
                              sem_g[k]).wait()

    def scatter(k, q):
        pltpu.async_copy(rows.at[k], acc.at[didx.at[q]], sem_s[k], add=True)

    def scatter_wait(k):
        pltpu.make_async_copy(rows.at[k], acc.at[didx.at[0]],
                              sem_s[k]).wait()

    def counts(q):
        for j in range(EB // L):
            dvec = didx[q, pl.ds(j * L, L)]
            plsc.addupdate_scatter(cnt_v, [dvec], ones16)

    for q in range(QN):
        idx_load(q, q)

    def group(g, refill):
        b0 = g * QN
        for k in (0, 1):
            idx_wait(k)
            gather(k, k)
        for k in (0, 1):
            gather_wait(k)
            scatter(k, k)
            counts(k)
        for k in (0, 1):
            idx_wait(k + 2)
            scatter_wait(k)
            gather(k, k + 2)
            if refill:
                idx_load(k, b0 + QN + k)
        for k in (0, 1):
            gather_wait(k)
            scatter(k, k + 2)
            counts(k + 2)
        for k in (0, 1):
            scatter_wait(k)
            if refill:
                idx_load(k + 2, b0 + QN + k + 2)

    def body(g, carry):
        group(g, True)
        return carry

    lax.fori_loop(0, G4 - 1, body, 0)
    group(G4 - 1, False)

    plsc.subcore_barrier()
    r0 = s * RPT
    pltpu.sync_copy(acc.at[pl.ds(r0, RPT)], aggr_out.at[c, pl.ds(r0, RPT)])
    pltpu.sync_copy(cnt_v, cnt_out.at[wid])


@functools.partial(
    pl.kernel,
    out_type=[jax.ShapeDtypeStruct((NW, 2 * N_PAD), jnp.float32)],
    mesh=_MESH,
    scratch_types=[
        pltpu.VMEM((TB, EB), jnp.int32),
        pltpu.VMEM((TB, EB), jnp.int32),
        pltpu.VMEM((N_PAD,), jnp.float32),
        pltpu.VMEM((N_PAD,), jnp.float32),
        pltpu.VMEM((N_PAD,), jnp.float32),
        pltpu.VMEM((N_PAD,), jnp.float32),
    ],
    compiler_params=_SC_PARAMS,
)
def _sc_edge2(p0_hbm, p1_hbm, src_hbm, dst_hbm, zcnt_hbm,
              out, src_all, dst_all, p0_v, p1_v, a0_v, a1_v):
    c = lax.axis_index("c")
    s = lax.axis_index("s")
    wid = c * NS + s
    pltpu.sync_copy(p0_hbm, p0_v)
    pltpu.sync_copy(p1_hbm, p1_v)
    pltpu.sync_copy(zcnt_hbm, a0_v)
    pltpu.sync_copy(zcnt_hbm, a1_v)
    pltpu.sync_copy(src_hbm.at[wid], src_all)
    pltpu.sync_copy(dst_hbm.at[wid], dst_all)

    def estep(b, carry):
        for j in range(EB // L):
            svec = src_all[b, pl.ds(j * L, L)]
            dvec = dst_all[b, pl.ds(j * L, L)]
            v0 = plsc.load_gather(p0_v, [svec])
            v1 = plsc.load_gather(p1_v, [svec])
            plsc.addupdate_scatter(a0_v, [dvec], v0)
            plsc.addupdate_scatter(a1_v, [dvec], v1)
        return carry

    lax.fori_loop(0, TB, estep, 0)
    pltpu.sync_copy(a0_v, out.at[wid, pl.ds(0, N_PAD)])
    pltpu.sync_copy(a1_v, out.at[wid, pl.ds(N_PAD, N_PAD)])


RB = 2048


def _tc_mid_body(aggr_ref, cnt_ref, x_ref, wl1_ref, wc_ref, bc_ref, g_ref,
                 bln_ref, w2_ref, p0_ref, p1_ref, r0_ref, r1_ref):
    cnt = jnp.maximum(jnp.sum(cnt_ref[...], axis=0), 1.0).reshape(-1, 1)
    mean = (aggr_ref[0] + aggr_ref[1]) / cnt
    x1 = (jnp.dot(mean, wl1_ref[...], preferred_element_type=jnp.float32)
          + jnp.dot(x_ref[...], wc_ref[...], preferred_element_type=jnp.float32)
          + bc_ref[...])
    mu = jnp.mean(x1, axis=-1, keepdims=True)
    var = jnp.mean((x1 - mu) * (x1 - mu), axis=-1, keepdims=True)
    xn = (x1 - mu) * lax.rsqrt(var + 1e-5) * g_ref[...] + bln_ref[...]
    h = jnp.where(xn > 0, xn, jnp.exp(jnp.minimum(xn, 0.0)) - 1.0)
    pr = jnp.dot(h, w2_ref[...], preferred_element_type=jnp.float32)
    p0_ref[...] = pr[:, 0]
    p1_ref[...] = pr[:, 1]
    r0_ref[...] = pr[:, 2]
    r1_ref[...] = pr[:, 3]


def _tc_out_body(a0_ref, a1_ref, cnt_ref, r0_ref, r1_ref, b2_ref,
                 o0_ref, o1_ref):
    cnt = jnp.maximum(jnp.sum(cnt_ref[...], axis=0), 1.0)
    o0_ref[...] = (jnp.sum(a0_ref[...], axis=0) / cnt + r0_ref[...]
                   + b2_ref[0, 0])
    o1_ref[...] = (jnp.sum(a1_ref[...], axis=0) / cnt + r1_ref[...]
                   + b2_ref[0, 1])


def kernel(x, edge_index, Wl1, bl1, Wr1, Ws, bs, g1, b1, Wl2, bl2, Wr2):
    src = edge_index[0]
    dst = edge_index[1]
    pad = E_PAD - E
    srcp = jnp.concatenate([src, jnp.zeros((pad,), jnp.int32)]).reshape(NW, TB, EB)
    dstp = jnp.concatenate([dst, jnp.full((pad,), N, jnp.int32)]).reshape(NW, TB, EB)
    zrow = jnp.zeros((RPT, D), jnp.float32)
    zcnt = jnp.zeros((N_PAD,), jnp.float32)

    aggr_p, cnt_p = _sc_segsum(x, srcp, dstp, zrow, zcnt)

    xp = jnp.concatenate([x, jnp.zeros((N_PAD - N, D), jnp.float32)])
    wl1t = Wl1.T
    wct = (Wr1 + Ws).T
    bc = (bl1 + bs).reshape(1, H)
    g = g1.reshape(1, H)
    bln = b1.reshape(1, H)
    w2 = jnp.zeros((H, D), jnp.float32)
    w2 = w2.at[:, 0:2].set(Wl2.T).at[:, 2:4].set(Wr2.T)
    b2 = bl2.reshape(1, OUT)

    grid = (N_PAD // RB,)
    p0, p1, r0, r1 = pl.pallas_call(
        _tc_mid_body,
        grid=grid,
        in_specs=[
            pl.BlockSpec((NC, RB, D), lambda i: (0, i, 0)),
            pl.BlockSpec((NW, RB), lambda i: (0, i)),
            pl.BlockSpec((RB, D), lambda i: (i, 0)),
            pl.BlockSpec((D, H), lambda i: (0, 0)),
            pl.BlockSpec((D, H), lambda i: (0, 0)),
            pl.BlockSpec((1, H), lambda i: (0, 0)),
            pl.BlockSpec((1, H), lambda i: (0, 0)),
            pl.BlockSpec((1, H), lambda i: (0, 0)),
            pl.BlockSpec((H, D), lambda i: (0, 0)),
        ],
        out_specs=[
            pl.BlockSpec((RB,), lambda i: (i,)),
            pl.BlockSpec((RB,), lambda i: (i,)),
            pl.BlockSpec((RB,), lambda i: (i,)),
            pl.BlockSpec((RB,), lambda i: (i,)),
        ],
        out_shape=[jax.ShapeDtypeStruct((N_PAD,), jnp.float32)] * 4,
    )(aggr_p, cnt_p, xp, wl1t, wct, bc, g, bln, w2)

    (acc2_p,) = _sc_edge2(p0, p1, srcp, dstp, zcnt)

    NPB = N_PAD // RB
    o0, o1 = pl.pallas_call(
        _tc_out_body,
        grid=grid,
        in_specs=[
            pl.BlockSpec((NW, RB), lambda i: (0, i)),
            pl.BlockSpec((NW, RB), lambda i: (0, i + NPB)),
            pl.BlockSpec((NW, RB), lambda i: (0, i)),
            pl.BlockSpec((RB,), lambda i: (i,)),
            pl.BlockSpec((RB,), lambda i: (i,)),
            pl.BlockSpec((1, OUT), lambda i: (0, 0)),
        ],
        out_specs=[
            pl.BlockSpec((RB,), lambda i: (i,)),
            pl.BlockSpec((RB,), lambda i: (i,)),
        ],
        out_shape=[jax.ShapeDtypeStruct((N_PAD,), jnp.float32)] * 2,
    )(acc2_p, acc2_p, cnt_p, r0, r1, b2)

    return jnp.stack([o0[:N], o1[:N]], axis=1)


# speedup vs baseline: 18.7216x; 2.5740x over previous
import functools

import jax
import jax.numpy as jnp
from jax import lax
from jax.experimental import pallas as pl
from jax.experimental.pallas import tpu as pltpu
from jax.experimental.pallas import tpu_sc as plsc

N = 10000
D = 128
H = 256
OUT = 2
E = 320000

NC = 2
NS = 16
NW = NC * NS
L = 16
EB = 128
NBUF = 4
TB = 80
G = TB // NBUF
EPT = TB * EB
E_PAD = EPT * NW
N_PAD = 10240
RPT = N_PAD // NS
CW = 16

_MESH = plsc.VectorSubcoreMesh(core_axis_name="c", subcore_axis_name="s",
                               num_cores=NC, num_subcores=NS)
_SC_PARAMS = pltpu.CompilerParams(needs_layout_passes=False)


QN = 4
G4 = TB // QN


@functools.partial(
    pl.kernel,
    out_type=[
        jax.ShapeDtypeStruct((NC, N_PAD, D), jnp.float32),
        jax.ShapeDtypeStruct((NW, N_PAD), jnp.float32),
    ],
    mesh=_MESH,
    scratch_types=[
        pltpu.VMEM((QN, EB), jnp.int32),
        pltpu.VMEM((QN, EB), jnp.int32),
        pltpu.VMEM((2, EB, D), jnp.float32),
        pltpu.VMEM((N_PAD,), jnp.float32),
        pltpu.VMEM_SHARED((N_PAD, D), jnp.float32),
        [pltpu.SemaphoreType.DMA] * QN,
        [pltpu.SemaphoreType.DMA] * 2,
        [pltpu.SemaphoreType.DMA] * 2,
    ],
    compiler_params=_SC_PARAMS,
)
def _sc_segsum(x_hbm, src_hbm, dst_hbm, zrow_hbm, zcnt_hbm,
               aggr_out, cnt_out, sidx, didx, rows, cnt_v, acc,
               sem_i, sem_g, sem_s):
    c = lax.axis_index("c")
    s = lax.axis_index("s")
    wid = c * NS + s
    pltpu.sync_copy(zrow_hbm, acc.at[pl.ds(s * RPT, RPT)])
    pltpu.sync_copy(zcnt_hbm, cnt_v)
    plsc.subcore_barrier()

    ones16 = jnp.ones((L,), jnp.float32)

    def idx_load(q, b):
        pltpu.async_copy(src_hbm.at[wid, b], sidx.at[q], sem_i[q])
        pltpu.async_copy(dst_hbm.at[wid, b], didx.at[q], sem_i[q])

    def idx_wait(q):
        pltpu.make_async_copy(src_hbm.at[wid, 0], sidx.at[q], sem_i[q]).wait()
        pltpu.make_async_copy(dst_hbm.at[wid, 0], didx.at[q], sem_i[q]).wait()

    def gather(k, q):
        pltpu.async_copy(x_hbm.at[sidx.at[q]], rows.at[k], sem_g[k])

    def gather_wait(k):
        pltpu.make_async_copy(x_hbm.at[sidx.at[0]], rows.at[k],
                              sem_g[k]).wait()

    def scatter(k, q):
        pltpu.async_copy(rows.at[k], acc.at[didx.at[q]], sem_s[k], add=True)

    def scatter_wait(k):
        pltpu.make_async_copy(rows.at[k], acc.at[didx.at[0]],
                              sem_s[k]).wait()

    def counts(q):
        for j in range(EB // L):
            dvec = didx[q, pl.ds(j * L, L)]
            plsc.addupdate_scatter(cnt_v, [dvec], ones16)

    for q in range(QN):
        idx_load(q, q)

    def group(g, refill):
        b0 = g * QN
        for k in (0, 1):
            idx_wait(k)
            gather(k, k)
        for k in (0, 1):
            gather_wait(k)
            scatter(k, k)
            counts(k)
        for k in (0, 1):
            idx_wait(k + 2)
            scatter_wait(k)
            gather(k, k + 2)
            if refill:
                idx_load(k, b0 + QN + k)
        for k in (0, 1):
            gather_wait(k)
            scatter(k, k + 2)
            counts(k + 2)
        for k in (0, 1):
            scatter_wait(k)
            if refill:
                idx_load(k + 2, b0 + QN + k + 2)

    def body(g, carry):
        group(g, True)
        return carry

    lax.fori_loop(0, G4 - 1, body, 0)
    group(G4 - 1, False)

    plsc.subcore_barrier()
    r0 = s * RPT
    pltpu.sync_copy(acc.at[pl.ds(r0, RPT)], aggr_out.at[c, pl.ds(r0, RPT)])
    pltpu.sync_copy(cnt_v, cnt_out.at[wid])


@functools.partial(
    pl.kernel,
    out_type=[jax.ShapeDtypeStruct((NW, 2 * N_PAD), jnp.float32)],
    mesh=_MESH,
    scratch_types=[
        pltpu.VMEM((TB, EB), jnp.int32),
        pltpu.VMEM((TB, EB), jnp.int32),
        pltpu.VMEM((N_PAD,), jnp.float32),
        pltpu.VMEM((N_PAD,), jnp.float32),
        pltpu.VMEM((N_PAD,), jnp.float32),
        pltpu.VMEM((N_PAD,), jnp.float32),
    ],
    compiler_params=_SC_PARAMS,
)
def _sc_edge2(p0_hbm, p1_hbm, src_hbm, dst_hbm, zcnt_hbm,
              out, src_all, dst_all, p0_v, p1_v, a0_v, a1_v):
    c = lax.axis_index("c")
    s = lax.axis_index("s")
    wid = c * NS + s
    pltpu.sync_copy(p0_hbm, p0_v)
    pltpu.sync_copy(p1_hbm, p1_v)
    pltpu.sync_copy(zcnt_hbm, a0_v)
    pltpu.sync_copy(zcnt_hbm, a1_v)
    pltpu.sync_copy(src_hbm.at[wid], src_all)
    pltpu.sync_copy(dst_hbm.at[wid], dst_all)

    def estep(b, carry):
        for j in range(EB // L):
            svec = src_all[b, pl.ds(j * L, L)]
            dvec = dst_all[b, pl.ds(j * L, L)]
            v0 = plsc.load_gather(p0_v, [svec])
            v1 = plsc.load_gather(p1_v, [svec])
            plsc.addupdate_scatter(a0_v, [dvec], v0)
            plsc.addupdate_scatter(a1_v, [dvec], v1)
        return carry

    lax.fori_loop(0, TB, estep, 0)
    pltpu.sync_copy(a0_v, out.at[wid, pl.ds(0, N_PAD)])
    pltpu.sync_copy(a1_v, out.at[wid, pl.ds(N_PAD, N_PAD)])


RB = 2048


def _tc_mid_body(aggr_ref, cnt_ref, x_ref, wl1_ref, wc_ref, bc_ref, g_ref,
                 bln_ref, w2_ref, p0_ref, p1_ref, r0_ref, r1_ref):
    cnt = jnp.maximum(jnp.sum(cnt_ref[...], axis=0), 1.0).reshape(-1, 1)
    mean = (aggr_ref[0] + aggr_ref[1]) / cnt
    x1 = (jnp.dot(mean, wl1_ref[...], preferred_element_type=jnp.float32)
          + jnp.dot(x_ref[...], wc_ref[...], preferred_element_type=jnp.float32)
          + bc_ref[...])
    mu = jnp.mean(x1, axis=-1, keepdims=True)
    var = jnp.mean((x1 - mu) * (x1 - mu), axis=-1, keepdims=True)
    xn = (x1 - mu) * lax.rsqrt(var + 1e-5) * g_ref[...] + bln_ref[...]
    h = jnp.where(xn > 0, xn, jnp.exp(jnp.minimum(xn, 0.0)) - 1.0)
    pr = jnp.dot(h, w2_ref[...], preferred_element_type=jnp.float32)
    p0_ref[...] = pr[:, 0]
    p1_ref[...] = pr[:, 1]
    r0_ref[...] = pr[:, 2]
    r1_ref[...] = pr[:, 3]


def _tc_out_body(a0_ref, a1_ref, cnt_ref, r0_ref, r1_ref, b2_ref,
                 o0_ref, o1_ref):
    cnt = jnp.maximum(jnp.sum(cnt_ref[...], axis=0), 1.0)
    o0_ref[...] = (jnp.sum(a0_ref[...], axis=0) / cnt + r0_ref[...]
                   + b2_ref[0, 0])
    o1_ref[...] = (jnp.sum(a1_ref[...], axis=0) / cnt + r1_ref[...]
                   + b2_ref[0, 1])


def kernel(x, edge_index, Wl1, bl1, Wr1, Ws, bs, g1, b1, Wl2, bl2, Wr2):
    src = edge_index[0]
    dst = edge_index[1]
    pad = E_PAD - E
    ar = jnp.arange(pad, dtype=jnp.int32)
    srcp = jnp.concatenate([src, ar % N]).reshape(NW, TB, EB)
    dstp = jnp.concatenate([dst, N + ar % (N_PAD - N)]).reshape(NW, TB, EB)
    zrow = jnp.zeros((RPT, D), jnp.float32)
    zcnt = jnp.zeros((N_PAD,), jnp.float32)

    aggr_p, cnt_p = _sc_segsum(x, srcp, dstp, zrow, zcnt)

    xp = jnp.concatenate([x, jnp.zeros((N_PAD - N, D), jnp.float32)])
    wl1t = Wl1.T
    wct = (Wr1 + Ws).T
    bc = (bl1 + bs).reshape(1, H)
    g = g1.reshape(1, H)
    bln = b1.reshape(1, H)
    w2 = jnp.zeros((H, D), jnp.float32)
    w2 = w2.at[:, 0:2].set(Wl2.T).at[:, 2:4].set(Wr2.T)
    b2 = bl2.reshape(1, OUT)

    grid = (N_PAD // RB,)
    p0, p1, r0, r1 = pl.pallas_call(
        _tc_mid_body,
        grid=grid,
        in_specs=[
            pl.BlockSpec((NC, RB, D), lambda i: (0, i, 0)),
            pl.BlockSpec((NW, RB), lambda i: (0, i)),
            pl.BlockSpec((RB, D), lambda i: (i, 0)),
            pl.BlockSpec((D, H), lambda i: (0, 0)),
            pl.BlockSpec((D, H), lambda i: (0, 0)),
            pl.BlockSpec((1, H), lambda i: (0, 0)),
            pl.BlockSpec((1, H), lambda i: (0, 0)),
            pl.BlockSpec((1, H), lambda i: (0, 0)),
            pl.BlockSpec((H, D), lambda i: (0, 0)),
        ],
        out_specs=[
            pl.BlockSpec((RB,), lambda i: (i,)),
            pl.BlockSpec((RB,), lambda i: (i,)),
            pl.BlockSpec((RB,), lambda i: (i,)),
            pl.BlockSpec((RB,), lambda i: (i,)),
        ],
        out_shape=[jax.ShapeDtypeStruct((N_PAD,), jnp.float32)] * 4,
    )(aggr_p, cnt_p, xp, wl1t, wct, bc, g, bln, w2)

    (acc2_p,) = _sc_edge2(p0, p1, srcp, dstp, zcnt)

    NPB = N_PAD // RB
    o0, o1 = pl.pallas_call(
        _tc_out_body,
        grid=grid,
        in_specs=[
            pl.BlockSpec((NW, RB), lambda i: (0, i)),
            pl.BlockSpec((NW, RB), lambda i: (0, i + NPB)),
            pl.BlockSpec((NW, RB), lambda i: (0, i)),
            pl.BlockSpec((RB,), lambda i: (i,)),
            pl.BlockSpec((RB,), lambda i: (i,)),
            pl.BlockSpec((1, OUT), lambda i: (0, 0)),
        ],
        out_specs=[
            pl.BlockSpec((RB,), lambda i: (i,)),
            pl.BlockSpec((RB,), lambda i: (i,)),
        ],
        out_shape=[jax.ShapeDtypeStruct((N_PAD,), jnp.float32)] * 2,
    )(acc2_p, acc2_p, cnt_p, r0, r1, b2)

    return jnp.stack([o0[:N], o1[:N]], axis=1)


# speedup vs baseline: 21.0768x; 1.1258x over previous
import functools

import jax
import jax.numpy as jnp
from jax import lax
from jax.experimental import pallas as pl
from jax.experimental.pallas import tpu as pltpu
from jax.experimental.pallas import tpu_sc as plsc

N = 10000
D = 128
H = 256
OUT = 2
E = 320000

NC = 2
NS = 16
NW = NC * NS
L = 16
EB = 64
TB = 160
EPT = TB * EB
E_PAD = EPT * NW
N_PAD = 10240
RPT = N_PAD // NS
CW = 16

_MESH = plsc.VectorSubcoreMesh(core_axis_name="c", subcore_axis_name="s",
                               num_cores=NC, num_subcores=NS)
_SC_PARAMS = pltpu.CompilerParams(needs_layout_passes=False)


NSLOT = 4
NQ = 8
G8 = TB // NQ


@functools.partial(
    pl.kernel,
    out_type=[
        jax.ShapeDtypeStruct((NC, N_PAD, D), jnp.float32),
        jax.ShapeDtypeStruct((NW, N_PAD), jnp.float32),
    ],
    mesh=_MESH,
    scratch_types=[
        pltpu.VMEM((NQ, EB), jnp.int32),
        pltpu.VMEM((NQ, EB), jnp.int32),
        pltpu.VMEM((NSLOT, EB, D), jnp.float32),
        pltpu.VMEM((N_PAD,), jnp.float32),
        pltpu.VMEM_SHARED((N_PAD, D), jnp.float32),
        [pltpu.SemaphoreType.DMA] * NQ,
        [pltpu.SemaphoreType.DMA] * NSLOT,
        [pltpu.SemaphoreType.DMA] * NSLOT,
    ],
    compiler_params=_SC_PARAMS,
)
def _sc_segsum(x_hbm, src_hbm, dst_hbm, zrow_hbm, zcnt_hbm,
               aggr_out, cnt_out, sidx, didx, rows, cnt_v, acc,
               sem_i, sem_g, sem_s):
    c = lax.axis_index("c")
    s = lax.axis_index("s")
    wid = c * NS + s
    pltpu.sync_copy(zrow_hbm, acc.at[pl.ds(s * RPT, RPT)])
    pltpu.sync_copy(zcnt_hbm, cnt_v)
    plsc.subcore_barrier()

    ones16 = jnp.ones((L,), jnp.float32)

    def idx_load(q, b):
        pltpu.async_copy(src_hbm.at[wid, b], sidx.at[q], sem_i[q])
        pltpu.async_copy(dst_hbm.at[wid, b], didx.at[q], sem_i[q])

    def idx_wait(q):
        pltpu.make_async_copy(src_hbm.at[wid, 0], sidx.at[q], sem_i[q]).wait()
        pltpu.make_async_copy(dst_hbm.at[wid, 0], didx.at[q], sem_i[q]).wait()

    def gather(k, q):
        pltpu.async_copy(x_hbm.at[sidx.at[q]], rows.at[k], sem_g[k])

    def gather_wait(k):
        pltpu.make_async_copy(x_hbm.at[sidx.at[0]], rows.at[k],
                              sem_g[k]).wait()

    def scatter(k, q):
        pltpu.async_copy(rows.at[k], acc.at[didx.at[q]], sem_s[k], add=True)

    def scatter_wait(k):
        pltpu.make_async_copy(rows.at[k], acc.at[didx.at[0]],
                              sem_s[k]).wait()

    def counts(q):
        for j in range(EB // L):
            dvec = didx[q, pl.ds(j * L, L)]
            plsc.addupdate_scatter(cnt_v, [dvec], ones16)

    def subgroup(b_base, qoff, do_sw, refill):
        for k in range(NSLOT):
            q = k + qoff
            idx_wait(q)
            if do_sw:
                scatter_wait(k)
            if refill:
                idx_load((q + NSLOT) % NQ, b_base + k + NSLOT)
            gather(k, q)
        for k in range(NSLOT):
            gather_wait(k)
            scatter(k, k + qoff)
            counts(k + qoff)

    for q in range(NSLOT):
        idx_load(q, q)
    subgroup(0, 0, False, True)
    subgroup(NSLOT, NSLOT, True, True)

    def body(g, carry):
        b0 = g * NQ
        subgroup(b0, 0, True, True)
        subgroup(b0 + NSLOT, NSLOT, True, True)
        return carry

    lax.fori_loop(1, G8 - 1, body, 0)
    b0 = (G8 - 1) * NQ
    subgroup(b0, 0, True, True)
    subgroup(b0 + NSLOT, NSLOT, True, False)
    for k in range(NSLOT):
        scatter_wait(k)

    plsc.subcore_barrier()
    r0 = s * RPT
    pltpu.sync_copy(acc.at[pl.ds(r0, RPT)], aggr_out.at[c, pl.ds(r0, RPT)])
    pltpu.sync_copy(cnt_v, cnt_out.at[wid])


@functools.partial(
    pl.kernel,
    out_type=[jax.ShapeDtypeStruct((NW, 2 * N_PAD), jnp.float32)],
    mesh=_MESH,
    scratch_types=[
        pltpu.VMEM((TB, EB), jnp.int32),
        pltpu.VMEM((TB, EB), jnp.int32),
        pltpu.VMEM((N_PAD,), jnp.float32),
        pltpu.VMEM((N_PAD,), jnp.float32),
        pltpu.VMEM((N_PAD,), jnp.float32),
        pltpu.VMEM((N_PAD,), jnp.float32),
    ],
    compiler_params=_SC_PARAMS,
)
def _sc_edge2(p0_hbm, p1_hbm, src_hbm, dst_hbm, zcnt_hbm,
              out, src_all, dst_all, p0_v, p1_v, a0_v, a1_v):
    c = lax.axis_index("c")
    s = lax.axis_index("s")
    wid = c * NS + s
    pltpu.sync_copy(p0_hbm, p0_v)
    pltpu.sync_copy(p1_hbm, p1_v)
    pltpu.sync_copy(zcnt_hbm, a0_v)
    pltpu.sync_copy(zcnt_hbm, a1_v)
    pltpu.sync_copy(src_hbm.at[wid], src_all)
    pltpu.sync_copy(dst_hbm.at[wid], dst_all)

    def estep(b, carry):
        for j in range(EB // L):
            svec = src_all[b, pl.ds(j * L, L)]
            dvec = dst_all[b, pl.ds(j * L, L)]
            v0 = plsc.load_gather(p0_v, [svec])
            v1 = plsc.load_gather(p1_v, [svec])
            plsc.addupdate_scatter(a0_v, [dvec], v0)
            plsc.addupdate_scatter(a1_v, [dvec], v1)
        return carry

    lax.fori_loop(0, TB, estep, 0)
    pltpu.sync_copy(a0_v, out.at[wid, pl.ds(0, N_PAD)])
    pltpu.sync_copy(a1_v, out.at[wid, pl.ds(N_PAD, N_PAD)])


RB = 2048


def _tc_mid_body(aggr_ref, cnt_ref, x_ref, wl1_ref, wc_ref, bc_ref, g_ref,
                 bln_ref, w2_ref, p0_ref, p1_ref, r0_ref, r1_ref):
    cnt = jnp.maximum(jnp.sum(cnt_ref[...], axis=0), 1.0).reshape(-1, 1)
    mean = (aggr_ref[0] + aggr_ref[1]) / cnt
    x1 = (jnp.dot(mean, wl1_ref[...], preferred_element_type=jnp.float32)
          + jnp.dot(x_ref[...], wc_ref[...], preferred_element_type=jnp.float32)
          + bc_ref[...])
    mu = jnp.mean(x1, axis=-1, keepdims=True)
    var = jnp.mean((x1 - mu) * (x1 - mu), axis=-1, keepdims=True)
    xn = (x1 - mu) * lax.rsqrt(var + 1e-5) * g_ref[...] + bln_ref[...]
    h = jnp.where(xn > 0, xn, jnp.exp(jnp.minimum(xn, 0.0)) - 1.0)
    pr = jnp.dot(h, w2_ref[...], preferred_element_type=jnp.float32)
    p0_ref[...] = pr[:, 0]
    p1_ref[...] = pr[:, 1]
    r0_ref[...] = pr[:, 2]
    r1_ref[...] = pr[:, 3]


def _tc_out_body(a0_ref, a1_ref, cnt_ref, r0_ref, r1_ref, b2_ref,
                 o0_ref, o1_ref):
    cnt = jnp.maximum(jnp.sum(cnt_ref[...], axis=0), 1.0)
    o0_ref[...] = (jnp.sum(a0_ref[...], axis=0) / cnt + r0_ref[...]
                   + b2_ref[0, 0])
    o1_ref[...] = (jnp.sum(a1_ref[...], axis=0) / cnt + r1_ref[...]
                   + b2_ref[0, 1])


def kernel(x, edge_index, Wl1, bl1, Wr1, Ws, bs, g1, b1, Wl2, bl2, Wr2):
    src = edge_index[0]
    dst = edge_index[1]
    pad = E_PAD - E
    ar = jnp.arange(pad, dtype=jnp.int32)
    srcp = jnp.concatenate([src, ar % N]).reshape(NW, TB, EB)
    dstp = jnp.concatenate([dst, N + ar % (N_PAD - N)]).reshape(NW, TB, EB)
    zrow = jnp.zeros((RPT, D), jnp.float32)
    zcnt = jnp.zeros((N_PAD,), jnp.float32)

    aggr_p, cnt_p = _sc_segsum(x, srcp, dstp, zrow, zcnt)

    xp = jnp.concatenate([x, jnp.zeros((N_PAD - N, D), jnp.float32)])
    wl1t = Wl1.T
    wct = (Wr1 + Ws).T
    bc = (bl1 + bs).reshape(1, H)
    g = g1.reshape(1, H)
    bln = b1.reshape(1, H)
    w2 = jnp.zeros((H, D), jnp.float32)
    w2 = w2.at[:, 0:2].set(Wl2.T).at[:, 2:4].set(Wr2.T)
    b2 = bl2.reshape(1, OUT)

    grid = (N_PAD // RB,)
    p0, p1, r0, r1 = pl.pallas_call(
        _tc_mid_body,
        grid=grid,
        in_specs=[
            pl.BlockSpec((NC, RB, D), lambda i: (0, i, 0)),
            pl.BlockSpec((NW, RB), lambda i: (0, i)),
            pl.BlockSpec((RB, D), lambda i: (i, 0)),
            pl.BlockSpec((D, H), lambda i: (0, 0)),
            pl.BlockSpec((D, H), lambda i: (0, 0)),
            pl.BlockSpec((1, H), lambda i: (0, 0)),
            pl.BlockSpec((1, H), lambda i: (0, 0)),
            pl.BlockSpec((1, H), lambda i: (0, 0)),
            pl.BlockSpec((H, D), lambda i: (0, 0)),
        ],
        out_specs=[
            pl.BlockSpec((RB,), lambda i: (i,)),
            pl.BlockSpec((RB,), lambda i: (i,)),
            pl.BlockSpec((RB,), lambda i: (i,)),
            pl.BlockSpec((RB,), lambda i: (i,)),
        ],
        out_shape=[jax.ShapeDtypeStruct((N_PAD,), jnp.float32)] * 4,
    )(aggr_p, cnt_p, xp, wl1t, wct, bc, g, bln, w2)

    (acc2_p,) = _sc_edge2(p0, p1, srcp, dstp, zcnt)

    NPB = N_PAD // RB
    o0, o1 = pl.pallas_call(
        _tc_out_body,
        grid=grid,
        in_specs=[
            pl.BlockSpec((NW, RB), lambda i: (0, i)),
            pl.BlockSpec((NW, RB), lambda i: (0, i + NPB)),
            pl.BlockSpec((NW, RB), lambda i: (0, i)),
            pl.BlockSpec((RB,), lambda i: (i,)),
            pl.BlockSpec((RB,), lambda i: (i,)),
            pl.BlockSpec((1, OUT), lambda i: (0, 0)),
        ],
        out_specs=[
            pl.BlockSpec((RB,), lambda i: (i,)),
            pl.BlockSpec((RB,), lambda i: (i,)),
        ],
        out_shape=[jax.ShapeDtypeStruct((N_PAD,), jnp.float32)] * 2,
    )(acc2_p, acc2_p, cnt_p, r0, r1, b2)

    return jnp.stack([o0[:N], o1[:N]], axis=1)


# speedup vs baseline: 21.6101x; 1.0253x over previous
import functools

import jax
import jax.numpy as jnp
from jax import lax
from jax.experimental import pallas as pl
from jax.experimental.pallas import tpu as pltpu
from jax.experimental.pallas import tpu_sc as plsc

N = 10000
D = 128
H = 256
OUT = 2
E = 320000

NC = 2
NS = 16
NW = NC * NS
L = 16
EB = 64
TB = 160
EPT = TB * EB
E_PAD = EPT * NW
N_PAD = 10240
RPT = N_PAD // NS
CW = 16

_MESH = plsc.VectorSubcoreMesh(core_axis_name="c", subcore_axis_name="s",
                               num_cores=NC, num_subcores=NS)
_SC_PARAMS = pltpu.CompilerParams(needs_layout_passes=False)


NSLOT = 4
NQ = 8
G8 = TB // NQ


@functools.partial(
    pl.kernel,
    out_type=[
        jax.ShapeDtypeStruct((NC, N_PAD, D), jnp.float32),
        jax.ShapeDtypeStruct((NW, N_PAD), jnp.float32),
    ],
    mesh=_MESH,
    scratch_types=[
        pltpu.VMEM((NQ, EB), jnp.int32),
        pltpu.VMEM((NQ, EB), jnp.int32),
        pltpu.VMEM((NSLOT, EB, D), jnp.float32),
        pltpu.VMEM((N_PAD,), jnp.float32),
        pltpu.VMEM_SHARED((N_PAD, D), jnp.float32),
        [pltpu.SemaphoreType.DMA] * NQ,
        [pltpu.SemaphoreType.DMA] * NSLOT,
        [pltpu.SemaphoreType.DMA] * NSLOT,
    ],
    compiler_params=_SC_PARAMS,
)
def _sc_segsum(x_hbm, src_hbm, dst_hbm, zrow_hbm, zcnt_hbm,
               aggr_out, cnt_out, sidx, didx, rows, cnt_v, acc,
               sem_i, sem_g, sem_s):
    c = lax.axis_index("c")
    s = lax.axis_index("s")
    wid = c * NS + s
    pltpu.sync_copy(zrow_hbm, acc.at[pl.ds(s * RPT, RPT)])
    pltpu.sync_copy(zcnt_hbm, cnt_v)
    plsc.subcore_barrier()

    ones16 = jnp.ones((L,), jnp.float32)

    def idx_load(q, b):
        pltpu.async_copy(src_hbm.at[wid, b], sidx.at[q], sem_i[q])
        pltpu.async_copy(dst_hbm.at[wid, b], didx.at[q], sem_i[q])

    def idx_wait(q):
        pltpu.make_async_copy(src_hbm.at[wid, 0], sidx.at[q], sem_i[q]).wait()
        pltpu.make_async_copy(dst_hbm.at[wid, 0], didx.at[q], sem_i[q]).wait()

    def gather(k, q):
        pltpu.async_copy(x_hbm.at[sidx.at[q]], rows.at[k], sem_g[k])

    def gather_wait(k):
        pltpu.make_async_copy(x_hbm.at[sidx.at[0]], rows.at[k],
                              sem_g[k]).wait()

    def scatter(k, q):
        pltpu.async_copy(rows.at[k], acc.at[didx.at[q]], sem_s[k], add=True)

    def scatter_wait(k):
        pltpu.make_async_copy(rows.at[k], acc.at[didx.at[0]],
                              sem_s[k]).wait()

    def counts(q):
        for j in range(EB // L):
            dvec = didx[q, pl.ds(j * L, L)]
            plsc.addupdate_scatter(cnt_v, [dvec], ones16)

    def subgroup(b_base, qoff, do_sw, refill):
        for k in range(NSLOT):
            q = k + qoff
            idx_wait(q)
            if do_sw:
                scatter_wait(k)
            if refill:
                idx_load((q + NSLOT) % NQ, b_base + k + NSLOT)
            gather(k, q)
        for k in range(NSLOT):
            gather_wait(k)
            scatter(k, k + qoff)
            counts(k + qoff)

    for q in range(NSLOT):
        idx_load(q, q)
    subgroup(0, 0, False, True)
    subgroup(NSLOT, NSLOT, True, True)

    def body(g, carry):
        b0 = g * NQ
        subgroup(b0, 0, True, True)
        subgroup(b0 + NSLOT, NSLOT, True, True)
        return carry

    lax.fori_loop(1, G8 - 1, body, 0)
    b0 = (G8 - 1) * NQ
    subgroup(b0, 0, True, True)
    subgroup(b0 + NSLOT, NSLOT, True, False)
    for k in range(NSLOT):
        scatter_wait(k)

    plsc.subcore_barrier()
    r0 = s * RPT
    pltpu.sync_copy(acc.at[pl.ds(r0, RPT)], aggr_out.at[c, pl.ds(r0, RPT)])
    pltpu.sync_copy(cnt_v, cnt_out.at[wid])


@functools.partial(
    pl.kernel,
    out_type=[jax.ShapeDtypeStruct((NW, 2 * N_PAD), jnp.float32)],
    mesh=_MESH,
    scratch_types=[
        pltpu.VMEM((TB, EB), jnp.int32),
        pltpu.VMEM((TB, EB), jnp.int32),
        pltpu.VMEM((N_PAD,), jnp.float32),
        pltpu.VMEM((N_PAD,), jnp.float32),
        pltpu.VMEM((N_PAD,), jnp.float32),
        pltpu.VMEM((N_PAD,), jnp.float32),
    ],
    compiler_params=_SC_PARAMS,
)
def _sc_edge2(p0_hbm, p1_hbm, src_hbm, dst_hbm, zcnt_hbm,
              out, src_all, dst_all, p0_v, p1_v, a0_v, a1_v):
    c = lax.axis_index("c")
    s = lax.axis_index("s")
    wid = c * NS + s
    pltpu.sync_copy(p0_hbm, p0_v)
    pltpu.sync_copy(p1_hbm, p1_v)
    pltpu.sync_copy(zcnt_hbm, a0_v)
    pltpu.sync_copy(zcnt_hbm, a1_v)
    pltpu.sync_copy(src_hbm.at[wid], src_all)
    pltpu.sync_copy(dst_hbm.at[wid], dst_all)

    def estep(b, carry):
        for j in range(EB // L):
            svec = src_all[b, pl.ds(j * L, L)]
            dvec = dst_all[b, pl.ds(j * L, L)]
            v0 = plsc.load_gather(p0_v, [svec])
            v1 = plsc.load_gather(p1_v, [svec])
            plsc.addupdate_scatter(a0_v, [dvec], v0)
            plsc.addupdate_scatter(a1_v, [dvec], v1)
        return carry

    lax.fori_loop(0, TB, estep, 0)
    pltpu.sync_copy(a0_v, out.at[wid, pl.ds(0, N_PAD)])
    pltpu.sync_copy(a1_v, out.at[wid, pl.ds(N_PAD, N_PAD)])


RB = 2048


def _tc_xw_body(x_ref, wc_ref, bc_ref, xw_ref):
    xw_ref[...] = (jnp.dot(x_ref[...], wc_ref[...],
                           preferred_element_type=jnp.float32) + bc_ref[...])


def _tc_mid_body(aggr_ref, cnt_ref, xw_ref, wl1_ref, g_ref,
                 bln_ref, w2_ref, p0_ref, p1_ref, r0_ref, r1_ref):
    cnt = jnp.maximum(jnp.sum(cnt_ref[...], axis=0), 1.0).reshape(-1, 1)
    mean = (aggr_ref[0] + aggr_ref[1]) / cnt
    x1 = (jnp.dot(mean, wl1_ref[...], preferred_element_type=jnp.float32)
          + xw_ref[...])
    mu = jnp.mean(x1, axis=-1, keepdims=True)
    var = jnp.mean((x1 - mu) * (x1 - mu), axis=-1, keepdims=True)
    xn = (x1 - mu) * lax.rsqrt(var + 1e-5) * g_ref[...] + bln_ref[...]
    h = jnp.where(xn > 0, xn, jnp.exp(jnp.minimum(xn, 0.0)) - 1.0)
    pr = jnp.dot(h, w2_ref[...], preferred_element_type=jnp.float32)
    p0_ref[...] = pr[:, 0]
    p1_ref[...] = pr[:, 1]
    r0_ref[...] = pr[:, 2]
    r1_ref[...] = pr[:, 3]


def _tc_out_body(a0_ref, a1_ref, cnt_ref, r0_ref, r1_ref, b2_ref,
                 o0_ref, o1_ref):
    cnt = jnp.maximum(jnp.sum(cnt_ref[...], axis=0), 1.0)
    o0_ref[...] = (jnp.sum(a0_ref[...], axis=0) / cnt + r0_ref[...]
                   + b2_ref[0, 0])
    o1_ref[...] = (jnp.sum(a1_ref[...], axis=0) / cnt + r1_ref[...]
                   + b2_ref[0, 1])


def kernel(x, edge_index, Wl1, bl1, Wr1, Ws, bs, g1, b1, Wl2, bl2, Wr2):
    src = edge_index[0]
    dst = edge_index[1]
    pad = E_PAD - E
    ar = jnp.arange(pad, dtype=jnp.int32)
    srcp = jnp.concatenate([src, ar % N]).reshape(NW, TB, EB)
    dstp = jnp.concatenate([dst, N + ar % (N_PAD - N)]).reshape(NW, TB, EB)
    zrow = jnp.zeros((RPT, D), jnp.float32)
    zcnt = jnp.zeros((N_PAD,), jnp.float32)

    xp = jnp.concatenate([x, jnp.zeros((N_PAD - N, D), jnp.float32)])
    wl1t = Wl1.T
    wct = (Wr1 + Ws).T
    bc = (bl1 + bs).reshape(1, H)
    g = g1.reshape(1, H)
    bln = b1.reshape(1, H)
    w2 = jnp.zeros((H, D), jnp.float32)
    w2 = w2.at[:, 0:2].set(Wl2.T).at[:, 2:4].set(Wr2.T)
    b2 = bl2.reshape(1, OUT)

    grid = (N_PAD // RB,)
    xw = pl.pallas_call(
        _tc_xw_body,
        grid=grid,
        in_specs=[
            pl.BlockSpec((RB, D), lambda i: (i, 0)),
            pl.BlockSpec((D, H), lambda i: (0, 0)),
            pl.BlockSpec((1, H), lambda i: (0, 0)),
        ],
        out_specs=pl.BlockSpec((RB, H), lambda i: (i, 0)),
        out_shape=jax.ShapeDtypeStruct((N_PAD, H), jnp.float32),
    )(xp, wct, bc)

    aggr_p, cnt_p = _sc_segsum(x, srcp, dstp, zrow, zcnt)

    p0, p1, r0, r1 = pl.pallas_call(
        _tc_mid_body,
        grid=grid,
        in_specs=[
            pl.BlockSpec((NC, RB, D), lambda i: (0, i, 0)),
            pl.BlockSpec((NW, RB), lambda i: (0, i)),
            pl.BlockSpec((RB, H), lambda i: (i, 0)),
            pl.BlockSpec((D, H), lambda i: (0, 0)),
            pl.BlockSpec((1, H), lambda i: (0, 0)),
            pl.BlockSpec((1, H), lambda i: (0, 0)),
            pl.BlockSpec((H, D), lambda i: (0, 0)),
        ],
        out_specs=[
            pl.BlockSpec((RB,), lambda i: (i,)),
            pl.BlockSpec((RB,), lambda i: (i,)),
            pl.BlockSpec((RB,), lambda i: (i,)),
            pl.BlockSpec((RB,), lambda i: (i,)),
        ],
        out_shape=[jax.ShapeDtypeStruct((N_PAD,), jnp.float32)] * 4,
    )(aggr_p, cnt_p, xw, wl1t, g, bln, w2)

    (acc2_p,) = _sc_edge2(p0, p1, srcp, dstp, zcnt)

    NPB = N_PAD // RB
    o0, o1 = pl.pallas_call(
        _tc_out_body,
        grid=grid,
        in_specs=[
            pl.BlockSpec((NW, RB), lambda i: (0, i)),
            pl.BlockSpec((NW, RB), lambda i: (0, i + NPB)),
            pl.BlockSpec((NW, RB), lambda i: (0, i)),
            pl.BlockSpec((RB,), lambda i: (i,)),
            pl.BlockSpec((RB,), lambda i: (i,)),
            pl.BlockSpec((1, OUT), lambda i: (0, 0)),
        ],
        out_specs=[
            pl.BlockSpec((RB,), lambda i: (i,)),
            pl.BlockSpec((RB,), lambda i: (i,)),
        ],
        out_shape=[jax.ShapeDtypeStruct((N_PAD,), jnp.float32)] * 2,
    )(acc2_p, acc2_p, cnt_p, r0, r1, b2)

    return jnp.stack([o0[:N], o1[:N]], axis=1)
